# Initial kernel scaffold; baseline (speedup 1.0000x reference)
import jax
import jax.numpy as jnp
from jax.experimental import pallas as pl


def kernel(x, edge_attr, lin0_0_w, lin0_0_b, lin0_1_w, lin0_1_b, em0_0_w, em0_0_b, em0_1_w, em0_1_b, emb0, ln0_g, ln0_b, lin1_0_w, lin1_0_b, lin1_1_w, lin1_1_b, em1_0_w, em1_0_b, em1_1_w, em1_1_b, emb1, ln1_g, ln1_b, fc_w, fc_b, edge_index, node_type, edge_type):
    raise NotImplementedError("write your pallas kernel here")


# speedup vs baseline: 5.1509x; 5.1509x over previous
import functools

import jax
import jax.numpy as jnp
from jax import lax
from jax.experimental import pallas as pl
from jax.experimental.pallas import tpu as pltpu
from jax.experimental.pallas import tpu_sc as plsc

_NC = 2
_NS = 16
_B = 80


def _softplus(x):
    return jnp.maximum(x, 0.0) + jnp.log(1.0 + jnp.exp(-jnp.abs(x)))


def _edge_prep_body(ea_t_ref, et_ref,
                    emb0t_ref, ew00t_ref, eb00_ref, ew01t_ref, eb01_ref,
                    emb1t_ref, ew10t_ref, eb10_ref, ew11t_ref, eb11_ref,
                    w0_ref, c0_ref, w1_ref, c1_ref):
    ea = ea_t_ref[...]
    et = et_ref[...]
    is_pump = et == 1
    direction = ea[-2:-1, :]
    pump_speed = ea[-1:, :]
    speed_scale = jnp.where(direction > 0, direction, jnp.ones_like(direction))
    speed_local = pump_speed * speed_scale
    sign = direction * 2.0 - 1.0

    layers = (
        (emb0t_ref, ew00t_ref, eb00_ref, ew01t_ref, eb01_ref, w0_ref, c0_ref),
        (emb1t_ref, ew10t_ref, eb10_ref, ew11t_ref, eb11_ref, w1_ref, c1_ref),
    )
    for embt_ref, ew0t_ref, eb0_ref, ew1t_ref, eb1_ref, w_ref, c_ref in layers:
        embt = embt_ref[...]
        eal = ea + jnp.where(is_pump, embt[:, 1:2], embt[:, 0:1])
        ew0t = ew0t_ref[...]
        ew1t = ew1t_ref[...]
        raw0 = jnp.sum(eal * ew0t[:, 0:1], axis=0, keepdims=True) + eb0_ref[0, 0]
        raw1a = jnp.sum(eal * ew1t[:, 0:1], axis=0, keepdims=True) + eb1_ref[0, 0]
        raw1b = jnp.sum(eal * ew1t[:, 1:2], axis=0, keepdims=True) + eb1_ref[0, 1]
        gain = jnp.where(is_pump, _softplus(raw1a) * speed_local, _softplus(raw0))
        bias = jnp.where(is_pump, raw1b * speed_local, jnp.zeros_like(raw0))
        w_ref[...] = sign * gain
        c_ref[...] = sign * bias


def _edge_prep(ea_t, et2, emb0t, ew00t, eb00, ew01t, eb01,
               emb1t, ew10t, eb10, ew11t, eb11):
    ED, E = ea_t.shape
    EB = 6400
    grid = (E // EB,)
    full = lambda shape: pl.BlockSpec(shape, lambda j: (0, 0))
    edge_vec = pl.BlockSpec((1, EB), lambda j: (0, j))
    return pl.pallas_call(
        _edge_prep_body,
        grid=grid,
        in_specs=[
            pl.BlockSpec((ED, EB), lambda j: (0, j)),
            edge_vec,
            full((ED, 2)), full((ED, 1)), full((1, 1)), full((ED, 2)), full((1, 2)),
            full((ED, 2)), full((ED, 1)), full((1, 1)), full((ED, 2)), full((1, 2)),
        ],
        out_specs=[edge_vec, edge_vec, edge_vec, edge_vec],
        out_shape=[jax.ShapeDtypeStruct((1, E), jnp.float32)] * 4,
    )(ea_t, et2, emb0t, ew00t, eb00, ew01t, eb01,
      emb1t, ew10t, eb10, ew11t, eb11)


def _sc_aggr_body(npad, E, D,
                  h_hbm, src_hbm, dst_hbm, w_hbm, z_rows_hbm,
                  aggr_out,
                  sidx, didx, wv, rows, aggr_sh, sem):
    c_id = lax.axis_index("c")
    s_id = lax.axis_index("s")
    wid = c_id * _NS + s_id
    rpt = npad // _NS
    epw = E // (_NC * _NS)
    nblk = epw // _B

    pltpu.sync_copy(z_rows_hbm, aggr_sh.at[pl.ds(s_id * rpt, rpt)])
    plsc.subcore_barrier()

    base0 = wid * epw
    ones = jnp.ones((16,), jnp.float32)

    def block(j, carry):
        base = base0 + j * _B
        pltpu.sync_copy(src_hbm.at[pl.ds(base, _B)], sidx)
        pltpu.sync_copy(dst_hbm.at[pl.ds(base, _B)], didx)
        pltpu.sync_copy(w_hbm.at[pl.ds(base, _B)], wv)
        pltpu.async_copy(h_hbm.at[sidx], rows, sem).wait()

        def group(g, carry2):
            base_r = g * 16
            wv16 = wv[pl.ds(base_r, 16)]
            for r in range(16):
                i = base_r + r
                wb = wv16[r] * ones
                for k in range(D // 16):
                    sl = pl.ds(k * 16, 16)
                    rows[i, sl] = rows[i, sl] * wb
            return carry2

        lax.fori_loop(0, _B // 16, group, 0)
        pltpu.sync_copy(rows, aggr_sh.at[didx], add=True)
        return carry

    lax.fori_loop(0, nblk, block, 0)
    plsc.subcore_barrier()

    sl = pl.ds(s_id * rpt, rpt)
    pltpu.sync_copy(aggr_sh.at[sl], aggr_out.at[c_id, sl])


def _sc_aggr(npad, h, src, dst, w, z_rows):
    N, D = h.shape
    E = src.shape[0]
    mesh = plsc.VectorSubcoreMesh(core_axis_name="c", subcore_axis_name="s",
                                  num_cores=_NC, num_subcores=_NS)
    kern = pl.kernel(
        functools.partial(_sc_aggr_body, npad, E, D),
        out_type=jax.ShapeDtypeStruct((_NC, npad, D), jnp.float32),
        mesh=mesh,
        scratch_types=[
            pltpu.VMEM((_B,), jnp.int32),
            pltpu.VMEM((_B,), jnp.int32),
            pltpu.VMEM((_B,), jnp.float32),
            pltpu.VMEM((_B, D), jnp.float32),
            pltpu.VMEM_SHARED((npad, D), jnp.float32),
            pltpu.SemaphoreType.DMA,
        ],
    )
    return kern(h, src, dst, w, z_rows)


def _sc_sums_body(npad, E, D,
                  dst_hbm, w0_hbm, c0_hbm, w1_hbm, c1_hbm, z_rows_hbm,
                  sums_out,
                  didx, w0v, c0v, w1v, c1v, wc, sums_sh, sem):
    c_id = lax.axis_index("c")
    s_id = lax.axis_index("s")
    wid = c_id * _NS + s_id
    rpt = npad // _NS
    epw = E // (_NC * _NS)
    nblk = epw // _B

    pltpu.sync_copy(z_rows_hbm, sums_sh.at[pl.ds(s_id * rpt, rpt)])
    plsc.subcore_barrier()

    zeros16 = jnp.zeros((16,), jnp.float32)

    def zrow(i, carry):
        for k in range(D // 16):
            wc[i, pl.ds(k * 16, 16)] = zeros16
        return carry

    lax.fori_loop(0, _B, zrow, 0)

    base0 = wid * epw
    lanes = lax.iota(jnp.int32, 16)
    ones = jnp.ones((16,), jnp.float32)

    def block(j, carry):
        base = base0 + j * _B
        pltpu.sync_copy(dst_hbm.at[pl.ds(base, _B)], didx)
        pltpu.sync_copy(w0_hbm.at[pl.ds(base, _B)], w0v)
        pltpu.sync_copy(c0_hbm.at[pl.ds(base, _B)], c0v)
        pltpu.sync_copy(w1_hbm.at[pl.ds(base, _B)], w1v)
        pltpu.sync_copy(c1_hbm.at[pl.ds(base, _B)], c1v)

        def group(g, carry2):
            base_r = g * 16
            w016 = w0v[pl.ds(base_r, 16)]
            c016 = c0v[pl.ds(base_r, 16)]
            w116 = w1v[pl.ds(base_r, 16)]
            c116 = c1v[pl.ds(base_r, 16)]
            for r in range(16):
                i = base_r + r
                row = jnp.where(lanes == 0, w016[r] * ones,
                                jnp.where(lanes == 1, c016[r] * ones,
                                          jnp.where(lanes == 2, w116[r] * ones,
                                                    jnp.where(lanes == 3, c116[r] * ones,
                                                              zeros16))))
                wc[i, pl.ds(0, 16)] = row
            return carry2

        lax.fori_loop(0, _B // 16, group, 0)
        pltpu.sync_copy(wc, sums_sh.at[didx], add=True)
        return carry

    lax.fori_loop(0, nblk, block, 0)
    plsc.subcore_barrier()

    sl = pl.ds(s_id * rpt, rpt)
    pltpu.sync_copy(sums_sh.at[sl], sums_out.at[c_id, sl])


def _sc_sums(npad, dst, w0, c0, w1, c1, z_rows):
    E = dst.shape[0]
    D = z_rows.shape[1]
    mesh = plsc.VectorSubcoreMesh(core_axis_name="c", subcore_axis_name="s",
                                  num_cores=_NC, num_subcores=_NS)
    kern = pl.kernel(
        functools.partial(_sc_sums_body, npad, E, D),
        out_type=jax.ShapeDtypeStruct((_NC, npad, D), jnp.float32),
        mesh=mesh,
        scratch_types=[
            pltpu.VMEM((_B,), jnp.int32),
            pltpu.VMEM((_B,), jnp.float32),
            pltpu.VMEM((_B,), jnp.float32),
            pltpu.VMEM((_B,), jnp.float32),
            pltpu.VMEM((_B,), jnp.float32),
            pltpu.VMEM((_B, D), jnp.float32),
            pltpu.VMEM_SHARED((npad, D), jnp.float32),
            pltpu.SemaphoreType.DMA,
        ],
    )
    return kern(dst, w0, c0, w1, c1, z_rows)


def _post_body(final, lane, ap_ref, sp_ref, h_ref, nt_ref,
               lw0t_ref, lb0_ref, lw1t_ref, lb1_ref, g_ref, b_ref,
               fcwt_ref, fcb_ref, out_ref):
    a = ap_ref[0] + ap_ref[1]
    s = sp_ref[0] + sp_ref[1]
    h = h_ref[...]
    aggr = a - s[:, lane:lane + 1] * h + s[:, lane + 1:lane + 2]
    o0 = jnp.dot(aggr, lw0t_ref[...], preferred_element_type=jnp.float32) + lb0_ref[...]
    o1 = jnp.dot(aggr, lw1t_ref[...], preferred_element_type=jnp.float32) + lb1_ref[...]
    o = jnp.where(nt_ref[...] == 1, o1, o0)
    mu = jnp.mean(o, axis=-1, keepdims=True)
    var = jnp.mean(jnp.square(o - mu), axis=-1, keepdims=True)
    o = (o - mu) / jnp.sqrt(var + 1e-5) * g_ref[...] + b_ref[...]
    o = jnp.maximum(o, 0.0) + h
    if final:
        o = jnp.dot(o, fcwt_ref[...], preferred_element_type=jnp.float32) + fcb_ref[...]
    out_ref[...] = o


def _post(final, lane, ap, sp, h, nt2, lw0t, lb0, lw1t, lb1, g, b, fcwt, fcb):
    N, D = h.shape
    R = 2000
    grid = (N // R,)
    full = lambda shape: pl.BlockSpec(shape, lambda j: (0, 0))
    return pl.pallas_call(
        functools.partial(_post_body, final, lane),
        grid=grid,
        in_specs=[
            pl.BlockSpec((_NC, R, D), lambda j: (0, j, 0)),
            pl.BlockSpec((_NC, R, D), lambda j: (0, j, 0)),
            pl.BlockSpec((R, D), lambda j: (j, 0)),
            pl.BlockSpec((R, 1), lambda j: (j, 0)),
            full((D, D)), full((1, D)), full((D, D)), full((1, D)),
            full((1, D)), full((1, D)), full((D, D)), full((1, D)),
        ],
        out_specs=pl.BlockSpec((R, D), lambda j: (j, 0)),
        out_shape=jax.ShapeDtypeStruct((N, D), jnp.float32),
    )(ap, sp, h, nt2, lw0t, lb0, lw1t, lb1, g, b, fcwt, fcb)


def kernel(x, edge_attr,
           lin0_0_w, lin0_0_b, lin0_1_w, lin0_1_b,
           em0_0_w, em0_0_b, em0_1_w, em0_1_b, emb0, ln0_g, ln0_b,
           lin1_0_w, lin1_0_b, lin1_1_w, lin1_1_b,
           em1_0_w, em1_0_b, em1_1_w, em1_1_b, emb1, ln1_g, ln1_b,
           fc_w, fc_b,
           edge_index, node_type, edge_type):
    N, D = x.shape
    E = edge_index.shape[1]

    ea_t = edge_attr.T
    et2 = edge_type.reshape(1, E)
    w0, c0, w1, c1 = _edge_prep(
        ea_t, et2,
        emb0.T, em0_0_w.T, em0_0_b.reshape(1, 1), em0_1_w.T, em0_1_b.reshape(1, 2),
        emb1.T, em1_0_w.T, em1_0_b.reshape(1, 1), em1_1_w.T, em1_1_b.reshape(1, 2),
    )

    src = edge_index[0]
    dst = edge_index[1]
    npad = ((N + 8 * _NS - 1) // (8 * _NS)) * (8 * _NS)
    rpt = npad // _NS
    z_rows = jnp.zeros((rpt, D), jnp.float32)
    nt2 = node_type.reshape(N, 1)

    sp = _sc_sums(npad, dst, w0.reshape(E), c0.reshape(E),
                  w1.reshape(E), c1.reshape(E), z_rows)

    h = x
    ap = _sc_aggr(npad, h, src, dst, w0.reshape(E), z_rows)
    h = _post(False, 0, ap, sp, h, nt2,
              lin0_0_w.T, lin0_0_b.reshape(1, D), lin0_1_w.T, lin0_1_b.reshape(1, D),
              ln0_g.reshape(1, D), ln0_b.reshape(1, D),
              fc_w.T, fc_b.reshape(1, D))

    ap = _sc_aggr(npad, h, src, dst, w1.reshape(E), z_rows)
    out = _post(True, 2, ap, sp, h, nt2,
                lin1_0_w.T, lin1_0_b.reshape(1, D), lin1_1_w.T, lin1_1_b.reshape(1, D),
                ln1_g.reshape(1, D), ln1_b.reshape(1, D),
                fc_w.T, fc_b.reshape(1, D))
    return out


# speedup vs baseline: 6.0100x; 1.1668x over previous
import functools

import jax
import jax.numpy as jnp
from jax import lax
from jax.experimental import pallas as pl
from jax.experimental.pallas import tpu as pltpu
from jax.experimental.pallas import tpu_sc as plsc

_NC = 2
_NS = 16
_B = 80


def _softplus(x):
    return jnp.maximum(x, 0.0) + jnp.log(1.0 + jnp.exp(-jnp.abs(x)))


def _edge_prep_body(ea_t_ref, et_ref,
                    emb0t_ref, ew00t_ref, eb00_ref, ew01t_ref, eb01_ref,
                    emb1t_ref, ew10t_ref, eb10_ref, ew11t_ref, eb11_ref,
                    w0_ref, c0_ref, w1_ref, c1_ref):
    ea = ea_t_ref[...]
    et = et_ref[...]
    is_pump = et == 1
    direction = ea[-2:-1, :]
    pump_speed = ea[-1:, :]
    speed_scale = jnp.where(direction > 0, direction, jnp.ones_like(direction))
    speed_local = pump_speed * speed_scale
    sign = direction * 2.0 - 1.0

    layers = (
        (emb0t_ref, ew00t_ref, eb00_ref, ew01t_ref, eb01_ref, w0_ref, c0_ref),
        (emb1t_ref, ew10t_ref, eb10_ref, ew11t_ref, eb11_ref, w1_ref, c1_ref),
    )
    for embt_ref, ew0t_ref, eb0_ref, ew1t_ref, eb1_ref, w_ref, c_ref in layers:
        embt = embt_ref[...]
        eal = ea + jnp.where(is_pump, embt[:, 1:2], embt[:, 0:1])
        ew0t = ew0t_ref[...]
        ew1t = ew1t_ref[...]
        raw0 = jnp.sum(eal * ew0t[:, 0:1], axis=0, keepdims=True) + eb0_ref[0, 0]
        raw1a = jnp.sum(eal * ew1t[:, 0:1], axis=0, keepdims=True) + eb1_ref[0, 0]
        raw1b = jnp.sum(eal * ew1t[:, 1:2], axis=0, keepdims=True) + eb1_ref[0, 1]
        gain = jnp.where(is_pump, _softplus(raw1a) * speed_local, _softplus(raw0))
        bias = jnp.where(is_pump, raw1b * speed_local, jnp.zeros_like(raw0))
        w_ref[...] = sign * gain
        c_ref[...] = sign * bias


def _edge_prep(ea_t, et2, emb0t, ew00t, eb00, ew01t, eb01,
               emb1t, ew10t, eb10, ew11t, eb11):
    ED, E = ea_t.shape
    EB = 6400
    grid = (E // EB,)
    full = lambda shape: pl.BlockSpec(shape, lambda j: (0, 0))
    edge_vec = pl.BlockSpec((1, EB), lambda j: (0, j))
    return pl.pallas_call(
        _edge_prep_body,
        grid=grid,
        in_specs=[
            pl.BlockSpec((ED, EB), lambda j: (0, j)),
            edge_vec,
            full((ED, 2)), full((ED, 1)), full((1, 1)), full((ED, 2)), full((1, 2)),
            full((ED, 2)), full((ED, 1)), full((1, 1)), full((ED, 2)), full((1, 2)),
        ],
        out_specs=[edge_vec, edge_vec, edge_vec, edge_vec],
        out_shape=[jax.ShapeDtypeStruct((1, E), jnp.float32)] * 4,
    )(ea_t, et2, emb0t, ew00t, eb00, ew01t, eb01,
      emb1t, ew10t, eb10, ew11t, eb11)


def _sc_aggr_body(npad, E, D,
                  h_hbm, src_hbm, dst_hbm, w_hbm, z_rows_hbm,
                  aggr_out,
                  sidx0, sidx1, didx0, didx1, wv0, wv1, rows0, rows1, aggr_sh,
                  gsem0, gsem1, ssem0, ssem1):
    c_id = lax.axis_index("c")
    s_id = lax.axis_index("s")
    wid = c_id * _NS + s_id
    rpt = npad // _NS
    epw = E // (_NC * _NS)
    nblk = epw // _B

    pltpu.sync_copy(z_rows_hbm, aggr_sh.at[pl.ds(s_id * rpt, rpt)])
    plsc.subcore_barrier()

    base0 = wid * epw
    ones = jnp.ones((16,), jnp.float32)
    bufs = ((sidx0, didx0, wv0, rows0, gsem0, ssem0),
            (sidx1, didx1, wv1, rows1, gsem1, ssem1))

    def load_and_gather(j, p):
        sidx, didx, wv, rows, gsem, _ = bufs[p]
        base = base0 + j * _B
        pltpu.sync_copy(src_hbm.at[pl.ds(base, _B)], sidx)
        pltpu.sync_copy(dst_hbm.at[pl.ds(base, _B)], didx)
        pltpu.sync_copy(w_hbm.at[pl.ds(base, _B)], wv)
        pltpu.async_copy(h_hbm.at[sidx], rows, gsem)

    def scale(p):
        _, _, wv, rows, _, _ = bufs[p]

        def group(g, carry2):
            base_r = g * 16
            wv16 = wv[pl.ds(base_r, 16)]
            for r in range(16):
                i = base_r + r
                wb = wv16[r] * ones
                for k in range(D // 16):
                    sl = pl.ds(k * 16, 16)
                    rows[i, sl] = rows[i, sl] * wb
            return carry2

        lax.fori_loop(0, _B // 16, group, 0)

    def wait_gather(p):
        sidx, _, _, rows, gsem, _ = bufs[p]
        pltpu.make_async_copy(h_hbm.at[sidx], rows, gsem).wait()

    def start_scatter(p):
        _, didx, _, rows, _, ssem = bufs[p]
        pltpu.async_copy(rows, aggr_sh.at[didx], ssem, add=True)

    def wait_scatter(p):
        _, didx, _, rows, _, ssem = bufs[p]
        pltpu.make_async_copy(rows, aggr_sh.at[didx], ssem).wait()

    load_and_gather(0, 0)
    wait_gather(0)
    load_and_gather(1, 1)
    scale(0)
    start_scatter(0)

    def section(j, p):
        wait_gather(p)
        wait_scatter(1 - p)
        load_and_gather(j + 1, 1 - p)
        scale(p)
        start_scatter(p)

    def pair(jj, carry):
        section(2 * jj, 0)
        section(2 * jj + 1, 1)
        return carry

    section(1, 1)
    lax.fori_loop(1, (nblk - 1) // 2, pair, 0)
    wait_gather(0)
    wait_scatter(1)
    scale(0)
    start_scatter(0)
    wait_scatter(0)

    plsc.subcore_barrier()
    sl = pl.ds(s_id * rpt, rpt)
    pltpu.sync_copy(aggr_sh.at[sl], aggr_out.at[c_id, sl])


def _sc_aggr(npad, h, src, dst, w, z_rows):
    N, D = h.shape
    E = src.shape[0]
    mesh = plsc.VectorSubcoreMesh(core_axis_name="c", subcore_axis_name="s",
                                  num_cores=_NC, num_subcores=_NS)
    kern = pl.kernel(
        functools.partial(_sc_aggr_body, npad, E, D),
        out_type=jax.ShapeDtypeStruct((_NC, npad, D), jnp.float32),
        mesh=mesh,
        scratch_types=[
            pltpu.VMEM((_B,), jnp.int32),
            pltpu.VMEM((_B,), jnp.int32),
            pltpu.VMEM((_B,), jnp.int32),
            pltpu.VMEM((_B,), jnp.int32),
            pltpu.VMEM((_B,), jnp.float32),
            pltpu.VMEM((_B,), jnp.float32),
            pltpu.VMEM((_B, D), jnp.float32),
            pltpu.VMEM((_B, D), jnp.float32),
            pltpu.VMEM_SHARED((npad, D), jnp.float32),
            pltpu.SemaphoreType.DMA,
            pltpu.SemaphoreType.DMA,
            pltpu.SemaphoreType.DMA,
            pltpu.SemaphoreType.DMA,
        ],
    )
    return kern(h, src, dst, w, z_rows)


def _sc_sums_body(npad, E, D,
                  dst_hbm, w0_hbm, c0_hbm, w1_hbm, c1_hbm, z_rows_hbm,
                  sums_out,
                  didx, w0v, c0v, w1v, c1v, wc, sums_sh, sem):
    c_id = lax.axis_index("c")
    s_id = lax.axis_index("s")
    wid = c_id * _NS + s_id
    rpt = npad // _NS
    epw = E // (_NC * _NS)
    nblk = epw // _B

    pltpu.sync_copy(z_rows_hbm, sums_sh.at[pl.ds(s_id * rpt, rpt)])
    plsc.subcore_barrier()

    zeros16 = jnp.zeros((16,), jnp.float32)

    def zrow(i, carry):
        for k in range(D // 16):
            wc[i, pl.ds(k * 16, 16)] = zeros16
        return carry

    lax.fori_loop(0, _B, zrow, 0)

    base0 = wid * epw
    lanes = lax.iota(jnp.int32, 16)
    ones = jnp.ones((16,), jnp.float32)

    def block(j, carry):
        base = base0 + j * _B
        pltpu.sync_copy(dst_hbm.at[pl.ds(base, _B)], didx)
        pltpu.sync_copy(w0_hbm.at[pl.ds(base, _B)], w0v)
        pltpu.sync_copy(c0_hbm.at[pl.ds(base, _B)], c0v)
        pltpu.sync_copy(w1_hbm.at[pl.ds(base, _B)], w1v)
        pltpu.sync_copy(c1_hbm.at[pl.ds(base, _B)], c1v)

        def group(g, carry2):
            base_r = g * 16
            w016 = w0v[pl.ds(base_r, 16)]
            c016 = c0v[pl.ds(base_r, 16)]
            w116 = w1v[pl.ds(base_r, 16)]
            c116 = c1v[pl.ds(base_r, 16)]
            for r in range(16):
                i = base_r + r
                row = jnp.where(lanes == 0, w016[r] * ones,
                                jnp.where(lanes == 1, c016[r] * ones,
                                          jnp.where(lanes == 2, w116[r] * ones,
                                                    jnp.where(lanes == 3, c116[r] * ones,
                                                              zeros16))))
                wc[i, pl.ds(0, 16)] = row
            return carry2

        lax.fori_loop(0, _B // 16, group, 0)
        pltpu.sync_copy(wc, sums_sh.at[didx], add=True)
        return carry

    lax.fori_loop(0, nblk, block, 0)
    plsc.subcore_barrier()

    sl = pl.ds(s_id * rpt, rpt)
    pltpu.sync_copy(sums_sh.at[sl], sums_out.at[c_id, sl])


def _sc_sums(npad, dst, w0, c0, w1, c1, z_rows):
    E = dst.shape[0]
    D = z_rows.shape[1]
    mesh = plsc.VectorSubcoreMesh(core_axis_name="c", subcore_axis_name="s",
                                  num_cores=_NC, num_subcores=_NS)
    kern = pl.kernel(
        functools.partial(_sc_sums_body, npad, E, D),
        out_type=jax.ShapeDtypeStruct((_NC, npad, D), jnp.float32),
        mesh=mesh,
        scratch_types=[
            pltpu.VMEM((_B,), jnp.int32),
            pltpu.VMEM((_B,), jnp.float32),
            pltpu.VMEM((_B,), jnp.float32),
            pltpu.VMEM((_B,), jnp.float32),
            pltpu.VMEM((_B,), jnp.float32),
            pltpu.VMEM((_B, D), jnp.float32),
            pltpu.VMEM_SHARED((npad, D), jnp.float32),
            pltpu.SemaphoreType.DMA,
        ],
    )
    return kern(dst, w0, c0, w1, c1, z_rows)


def _post_body(final, lane, ap_ref, sp_ref, h_ref, nt_ref,
               lw0t_ref, lb0_ref, lw1t_ref, lb1_ref, g_ref, b_ref,
               fcwt_ref, fcb_ref, out_ref):
    a = ap_ref[0] + ap_ref[1]
    s = sp_ref[0] + sp_ref[1]
    h = h_ref[...]
    aggr = a - s[:, lane:lane + 1] * h + s[:, lane + 1:lane + 2]
    o0 = jnp.dot(aggr, lw0t_ref[...], preferred_element_type=jnp.float32) + lb0_ref[...]
    o1 = jnp.dot(aggr, lw1t_ref[...], preferred_element_type=jnp.float32) + lb1_ref[...]
    o = jnp.where(nt_ref[...] == 1, o1, o0)
    mu = jnp.mean(o, axis=-1, keepdims=True)
    var = jnp.mean(jnp.square(o - mu), axis=-1, keepdims=True)
    o = (o - mu) / jnp.sqrt(var + 1e-5) * g_ref[...] + b_ref[...]
    o = jnp.maximum(o, 0.0) + h
    if final:
        o = jnp.dot(o, fcwt_ref[...], preferred_element_type=jnp.float32) + fcb_ref[...]
    out_ref[...] = o


def _post(final, lane, ap, sp, h, nt2, lw0t, lb0, lw1t, lb1, g, b, fcwt, fcb):
    N, D = h.shape
    R = 2000
    grid = (N // R,)
    full = lambda shape: pl.BlockSpec(shape, lambda j: (0, 0))
    return pl.pallas_call(
        functools.partial(_post_body, final, lane),
        grid=grid,
        in_specs=[
            pl.BlockSpec((_NC, R, D), lambda j: (0, j, 0)),
            pl.BlockSpec((_NC, R, D), lambda j: (0, j, 0)),
            pl.BlockSpec((R, D), lambda j: (j, 0)),
            pl.BlockSpec((R, 1), lambda j: (j, 0)),
            full((D, D)), full((1, D)), full((D, D)), full((1, D)),
            full((1, D)), full((1, D)), full((D, D)), full((1, D)),
        ],
        out_specs=pl.BlockSpec((R, D), lambda j: (j, 0)),
        out_shape=jax.ShapeDtypeStruct((N, D), jnp.float32),
    )(ap, sp, h, nt2, lw0t, lb0, lw1t, lb1, g, b, fcwt, fcb)


def kernel(x, edge_attr,
           lin0_0_w, lin0_0_b, lin0_1_w, lin0_1_b,
           em0_0_w, em0_0_b, em0_1_w, em0_1_b, emb0, ln0_g, ln0_b,
           lin1_0_w, lin1_0_b, lin1_1_w, lin1_1_b,
           em1_0_w, em1_0_b, em1_1_w, em1_1_b, emb1, ln1_g, ln1_b,
           fc_w, fc_b,
           edge_index, node_type, edge_type):
    N, D = x.shape
    E = edge_index.shape[1]

    ea_t = edge_attr.T
    et2 = edge_type.reshape(1, E)
    w0, c0, w1, c1 = _edge_prep(
        ea_t, et2,
        emb0.T, em0_0_w.T, em0_0_b.reshape(1, 1), em0_1_w.T, em0_1_b.reshape(1, 2),
        emb1.T, em1_0_w.T, em1_0_b.reshape(1, 1), em1_1_w.T, em1_1_b.reshape(1, 2),
    )

    src = edge_index[0]
    dst = edge_index[1]
    npad = ((N + 8 * _NS - 1) // (8 * _NS)) * (8 * _NS)
    rpt = npad // _NS
    z_rows = jnp.zeros((rpt, D), jnp.float32)
    nt2 = node_type.reshape(N, 1)

    sp = _sc_sums(npad, dst, w0.reshape(E), c0.reshape(E),
                  w1.reshape(E), c1.reshape(E), z_rows)

    h = x
    ap = _sc_aggr(npad, h, src, dst, w0.reshape(E), z_rows)
    h = _post(False, 0, ap, sp, h, nt2,
              lin0_0_w.T, lin0_0_b.reshape(1, D), lin0_1_w.T, lin0_1_b.reshape(1, D),
              ln0_g.reshape(1, D), ln0_b.reshape(1, D),
              fc_w.T, fc_b.reshape(1, D))

    ap = _sc_aggr(npad, h, src, dst, w1.reshape(E), z_rows)
    out = _post(True, 2, ap, sp, h, nt2,
                lin1_0_w.T, lin1_0_b.reshape(1, D), lin1_1_w.T, lin1_1_b.reshape(1, D),
                ln1_g.reshape(1, D), ln1_b.reshape(1, D),
                fc_w.T, fc_b.reshape(1, D))
    return out


# speedup vs baseline: 6.2805x; 1.0450x over previous
import functools

import jax
import jax.numpy as jnp
from jax import lax
from jax.experimental import pallas as pl
from jax.experimental.pallas import tpu as pltpu
from jax.experimental.pallas import tpu_sc as plsc

_NC = 2
_NS = 16
_B = 80


def _softplus(x):
    return jnp.maximum(x, 0.0) + jnp.log(1.0 + jnp.exp(-jnp.abs(x)))


def _edge_prep_body(ea_t_ref, et_ref,
                    emb0t_ref, ew00t_ref, eb00_ref, ew01t_ref, eb01_ref,
                    emb1t_ref, ew10t_ref, eb10_ref, ew11t_ref, eb11_ref,
                    w0_ref, c0_ref, w1_ref, c1_ref):
    ea = ea_t_ref[...]
    et = et_ref[...]
    is_pump = et == 1
    direction = ea[-2:-1, :]
    pump_speed = ea[-1:, :]
    speed_scale = jnp.where(direction > 0, direction, jnp.ones_like(direction))
    speed_local = pump_speed * speed_scale
    sign = direction * 2.0 - 1.0

    layers = (
        (emb0t_ref, ew00t_ref, eb00_ref, ew01t_ref, eb01_ref, w0_ref, c0_ref),
        (emb1t_ref, ew10t_ref, eb10_ref, ew11t_ref, eb11_ref, w1_ref, c1_ref),
    )
    for embt_ref, ew0t_ref, eb0_ref, ew1t_ref, eb1_ref, w_ref, c_ref in layers:
        embt = embt_ref[...]
        eal = ea + jnp.where(is_pump, embt[:, 1:2], embt[:, 0:1])
        ew0t = ew0t_ref[...]
        ew1t = ew1t_ref[...]
        raw0 = jnp.sum(eal * ew0t[:, 0:1], axis=0, keepdims=True) + eb0_ref[0, 0]
        raw1a = jnp.sum(eal * ew1t[:, 0:1], axis=0, keepdims=True) + eb1_ref[0, 0]
        raw1b = jnp.sum(eal * ew1t[:, 1:2], axis=0, keepdims=True) + eb1_ref[0, 1]
        gain = jnp.where(is_pump, _softplus(raw1a) * speed_local, _softplus(raw0))
        bias = jnp.where(is_pump, raw1b * speed_local, jnp.zeros_like(raw0))
        w_ref[...] = sign * gain
        c_ref[...] = sign * bias


def _edge_prep(ea_t, et2, emb0t, ew00t, eb00, ew01t, eb01,
               emb1t, ew10t, eb10, ew11t, eb11):
    ED, E = ea_t.shape
    EB = 6400
    grid = (E // EB,)
    full = lambda shape: pl.BlockSpec(shape, lambda j: (0, 0))
    edge_vec = pl.BlockSpec((1, EB), lambda j: (0, j))
    return pl.pallas_call(
        _edge_prep_body,
        grid=grid,
        in_specs=[
            pl.BlockSpec((ED, EB), lambda j: (0, j)),
            edge_vec,
            full((ED, 2)), full((ED, 1)), full((1, 1)), full((ED, 2)), full((1, 2)),
            full((ED, 2)), full((ED, 1)), full((1, 1)), full((ED, 2)), full((1, 2)),
        ],
        out_specs=[edge_vec, edge_vec, edge_vec, edge_vec],
        out_shape=[jax.ShapeDtypeStruct((1, E), jnp.float32)] * 4,
    )(ea_t, et2, emb0t, ew00t, eb00, ew01t, eb01,
      emb1t, ew10t, eb10, ew11t, eb11)


def _sc_aggr_body(npad, E, D,
                  h_hbm, src_hbm, dst_hbm, w_hbm, z_rows_hbm,
                  aggr_out,
                  sidx0, sidx1, didx0, didx1, wv0, wv1, rows0, rows1, aggr_sh,
                  gsem0, gsem1, ssem0, ssem1):
    c_id = lax.axis_index("c")
    s_id = lax.axis_index("s")
    wid = c_id * _NS + s_id
    rpt = npad // _NS
    epw = E // (_NC * _NS)
    nblk = epw // _B

    pltpu.sync_copy(z_rows_hbm, aggr_sh.at[pl.ds(s_id * rpt, rpt)])
    plsc.subcore_barrier()

    base0 = wid * epw
    ones = jnp.ones((16,), jnp.float32)
    bufs = ((sidx0, didx0, wv0, rows0, gsem0, ssem0),
            (sidx1, didx1, wv1, rows1, gsem1, ssem1))

    def load_and_gather(j, p):
        sidx, didx, wv, rows, gsem, _ = bufs[p]
        base = base0 + j * _B
        pltpu.sync_copy(src_hbm.at[pl.ds(base, _B)], sidx)
        pltpu.sync_copy(dst_hbm.at[pl.ds(base, _B)], didx)
        pltpu.sync_copy(w_hbm.at[pl.ds(base, _B)], wv)
        pltpu.async_copy(h_hbm.at[sidx], rows, gsem)

    def scale(p):
        _, _, wv, rows, _, _ = bufs[p]

        def group(g, carry2):
            base_r = g * 16
            wv16 = wv[pl.ds(base_r, 16)]
            for r in range(16):
                i = base_r + r
                wb = wv16[r] * ones
                for k in range(D // 16):
                    sl = pl.ds(k * 16, 16)
                    rows[i, sl] = rows[i, sl] * wb
            return carry2

        lax.fori_loop(0, _B // 16, group, 0)

    def wait_gather(p):
        sidx, _, _, rows, gsem, _ = bufs[p]
        pltpu.make_async_copy(h_hbm.at[sidx], rows, gsem).wait()

    def start_scatter(p):
        _, didx, _, rows, _, ssem = bufs[p]
        pltpu.async_copy(rows, aggr_sh.at[didx], ssem, add=True)

    def wait_scatter(p):
        _, didx, _, rows, _, ssem = bufs[p]
        pltpu.make_async_copy(rows, aggr_sh.at[didx], ssem).wait()

    load_and_gather(0, 0)
    wait_gather(0)
    load_and_gather(1, 1)
    scale(0)
    start_scatter(0)

    def section(j, p):
        wait_gather(p)
        wait_scatter(1 - p)
        load_and_gather(j + 1, 1 - p)
        scale(p)
        start_scatter(p)

    def pair(jj, carry):
        section(2 * jj, 0)
        section(2 * jj + 1, 1)
        return carry

    section(1, 1)
    lax.fori_loop(1, (nblk - 1) // 2, pair, 0)
    wait_gather(0)
    wait_scatter(1)
    scale(0)
    start_scatter(0)
    wait_scatter(0)

    plsc.subcore_barrier()
    sl = pl.ds(s_id * rpt, rpt)
    pltpu.sync_copy(aggr_sh.at[sl], aggr_out.at[c_id, sl])


def _sc_aggr(npad, h, src, dst, w, z_rows):
    N, D = h.shape
    E = src.shape[0]
    mesh = plsc.VectorSubcoreMesh(core_axis_name="c", subcore_axis_name="s",
                                  num_cores=_NC, num_subcores=_NS)
    kern = pl.kernel(
        functools.partial(_sc_aggr_body, npad, E, D),
        out_type=jax.ShapeDtypeStruct((_NC, npad, D), jnp.float32),
        mesh=mesh,
        scratch_types=[
            pltpu.VMEM((_B,), jnp.int32),
            pltpu.VMEM((_B,), jnp.int32),
            pltpu.VMEM((_B,), jnp.int32),
            pltpu.VMEM((_B,), jnp.int32),
            pltpu.VMEM((_B,), jnp.float32),
            pltpu.VMEM((_B,), jnp.float32),
            pltpu.VMEM((_B, D), jnp.float32),
            pltpu.VMEM((_B, D), jnp.float32),
            pltpu.VMEM_SHARED((npad, D), jnp.float32),
            pltpu.SemaphoreType.DMA,
            pltpu.SemaphoreType.DMA,
            pltpu.SemaphoreType.DMA,
            pltpu.SemaphoreType.DMA,
        ],
    )
    return kern(h, src, dst, w, z_rows)


def _sc_sums_body(npad, E, D,
                  dst_hbm, w0_hbm, c0_hbm, w1_hbm, c1_hbm, z_rows_hbm,
                  sums_out,
                  didx0, didx1, w0v, c0v, w1v, c1v, wc0, wc1, sums_sh,
                  ssem0, ssem1):
    c_id = lax.axis_index("c")
    s_id = lax.axis_index("s")
    wid = c_id * _NS + s_id
    rpt = npad // _NS
    epw = E // (_NC * _NS)
    nblk = epw // _B

    pltpu.sync_copy(z_rows_hbm, sums_sh.at[pl.ds(s_id * rpt, rpt)])
    plsc.subcore_barrier()

    zeros16 = jnp.zeros((16,), jnp.float32)

    def zrow(i, carry):
        for k in range(D // 16):
            wc0[i, pl.ds(k * 16, 16)] = zeros16
            wc1[i, pl.ds(k * 16, 16)] = zeros16
        return carry

    lax.fori_loop(0, _B, zrow, 0)

    base0 = wid * epw
    lanes = lax.iota(jnp.int32, 16)
    ones = jnp.ones((16,), jnp.float32)
    bufs = ((didx0, wc0, ssem0), (didx1, wc1, ssem1))

    def build(j, p):
        didx, wc, _ = bufs[p]
        base = base0 + j * _B
        pltpu.sync_copy(dst_hbm.at[pl.ds(base, _B)], didx)
        pltpu.sync_copy(w0_hbm.at[pl.ds(base, _B)], w0v)
        pltpu.sync_copy(c0_hbm.at[pl.ds(base, _B)], c0v)
        pltpu.sync_copy(w1_hbm.at[pl.ds(base, _B)], w1v)
        pltpu.sync_copy(c1_hbm.at[pl.ds(base, _B)], c1v)

        def group(g, carry2):
            base_r = g * 16
            w016 = w0v[pl.ds(base_r, 16)]
            c016 = c0v[pl.ds(base_r, 16)]
            w116 = w1v[pl.ds(base_r, 16)]
            c116 = c1v[pl.ds(base_r, 16)]
            for r in range(16):
                i = base_r + r
                row = jnp.where(lanes == 0, w016[r] * ones,
                                jnp.where(lanes == 1, c016[r] * ones,
                                          jnp.where(lanes == 2, w116[r] * ones,
                                                    jnp.where(lanes == 3, c116[r] * ones,
                                                              zeros16))))
                wc[i, pl.ds(0, 16)] = row
            return carry2

        lax.fori_loop(0, _B // 16, group, 0)

    def start_scatter(p):
        didx, wc, ssem = bufs[p]
        pltpu.async_copy(wc, sums_sh.at[didx], ssem, add=True)

    def wait_scatter(p):
        didx, wc, ssem = bufs[p]
        pltpu.make_async_copy(wc, sums_sh.at[didx], ssem).wait()

    build(0, 0)
    start_scatter(0)
    build(1, 1)
    start_scatter(1)

    def section(j, p):
        wait_scatter(p)
        build(j, p)
        start_scatter(p)

    def pair(jj, carry):
        section(2 * jj, 0)
        section(2 * jj + 1, 1)
        return carry

    lax.fori_loop(1, (nblk - 1) // 2, pair, 0)
    section(nblk - 1, 0)
    wait_scatter(0)
    wait_scatter(1)

    plsc.subcore_barrier()
    sl = pl.ds(s_id * rpt, rpt)
    pltpu.sync_copy(sums_sh.at[sl], sums_out.at[c_id, sl])


def _sc_sums(npad, dst, w0, c0, w1, c1, z_rows):
    E = dst.shape[0]
    D = z_rows.shape[1]
    mesh = plsc.VectorSubcoreMesh(core_axis_name="c", subcore_axis_name="s",
                                  num_cores=_NC, num_subcores=_NS)
    kern = pl.kernel(
        functools.partial(_sc_sums_body, npad, E, D),
        out_type=jax.ShapeDtypeStruct((_NC, npad, D), jnp.float32),
        mesh=mesh,
        scratch_types=[
            pltpu.VMEM((_B,), jnp.int32),
            pltpu.VMEM((_B,), jnp.int32),
            pltpu.VMEM((_B,), jnp.float32),
            pltpu.VMEM((_B,), jnp.float32),
            pltpu.VMEM((_B,), jnp.float32),
            pltpu.VMEM((_B,), jnp.float32),
            pltpu.VMEM((_B, D), jnp.float32),
            pltpu.VMEM((_B, D), jnp.float32),
            pltpu.VMEM_SHARED((npad, D), jnp.float32),
            pltpu.SemaphoreType.DMA,
            pltpu.SemaphoreType.DMA,
        ],
    )
    return kern(dst, w0, c0, w1, c1, z_rows)


def _post_body(final, lane, ap_ref, sp_ref, h_ref, nt_ref,
               lw0t_ref, lb0_ref, lw1t_ref, lb1_ref, g_ref, b_ref,
               fcwt_ref, fcb_ref, out_ref):
    a = ap_ref[0] + ap_ref[1]
    s = sp_ref[0] + sp_ref[1]
    h = h_ref[...]
    aggr = a - s[:, lane:lane + 1] * h + s[:, lane + 1:lane + 2]
    o0 = jnp.dot(aggr, lw0t_ref[...], preferred_element_type=jnp.float32) + lb0_ref[...]
    o1 = jnp.dot(aggr, lw1t_ref[...], preferred_element_type=jnp.float32) + lb1_ref[...]
    o = jnp.where(nt_ref[...] == 1, o1, o0)
    mu = jnp.mean(o, axis=-1, keepdims=True)
    var = jnp.mean(jnp.square(o - mu), axis=-1, keepdims=True)
    o = (o - mu) / jnp.sqrt(var + 1e-5) * g_ref[...] + b_ref[...]
    o = jnp.maximum(o, 0.0) + h
    if final:
        o = jnp.dot(o, fcwt_ref[...], preferred_element_type=jnp.float32) + fcb_ref[...]
    out_ref[...] = o


def _post(final, lane, ap, sp, h, nt2, lw0t, lb0, lw1t, lb1, g, b, fcwt, fcb):
    N, D = h.shape
    R = 2000
    grid = (N // R,)
    full = lambda shape: pl.BlockSpec(shape, lambda j: (0, 0))
    return pl.pallas_call(
        functools.partial(_post_body, final, lane),
        grid=grid,
        in_specs=[
            pl.BlockSpec((_NC, R, D), lambda j: (0, j, 0)),
            pl.BlockSpec((_NC, R, D), lambda j: (0, j, 0)),
            pl.BlockSpec((R, D), lambda j: (j, 0)),
            pl.BlockSpec((R, 1), lambda j: (j, 0)),
            full((D, D)), full((1, D)), full((D, D)), full((1, D)),
            full((1, D)), full((1, D)), full((D, D)), full((1, D)),
        ],
        out_specs=pl.BlockSpec((R, D), lambda j: (j, 0)),
        out_shape=jax.ShapeDtypeStruct((N, D), jnp.float32),
    )(ap, sp, h, nt2, lw0t, lb0, lw1t, lb1, g, b, fcwt, fcb)


def kernel(x, edge_attr,
           lin0_0_w, lin0_0_b, lin0_1_w, lin0_1_b,
           em0_0_w, em0_0_b, em0_1_w, em0_1_b, emb0, ln0_g, ln0_b,
           lin1_0_w, lin1_0_b, lin1_1_w, lin1_1_b,
           em1_0_w, em1_0_b, em1_1_w, em1_1_b, emb1, ln1_g, ln1_b,
           fc_w, fc_b,
           edge_index, node_type, edge_type):
    N, D = x.shape
    E = edge_index.shape[1]

    ea_t = edge_attr.T
    et2 = edge_type.reshape(1, E)
    w0, c0, w1, c1 = _edge_prep(
        ea_t, et2,
        emb0.T, em0_0_w.T, em0_0_b.reshape(1, 1), em0_1_w.T, em0_1_b.reshape(1, 2),
        emb1.T, em1_0_w.T, em1_0_b.reshape(1, 1), em1_1_w.T, em1_1_b.reshape(1, 2),
    )

    src = edge_index[0]
    dst = edge_index[1]
    npad = ((N + 8 * _NS - 1) // (8 * _NS)) * (8 * _NS)
    rpt = npad // _NS
    z_rows = jnp.zeros((rpt, D), jnp.float32)
    nt2 = node_type.reshape(N, 1)

    sp = _sc_sums(npad, dst, w0.reshape(E), c0.reshape(E),
                  w1.reshape(E), c1.reshape(E), z_rows)

    h = x
    ap = _sc_aggr(npad, h, src, dst, w0.reshape(E), z_rows)
    h = _post(False, 0, ap, sp, h, nt2,
              lin0_0_w.T, lin0_0_b.reshape(1, D), lin0_1_w.T, lin0_1_b.reshape(1, D),
              ln0_g.reshape(1, D), ln0_b.reshape(1, D),
              fc_w.T, fc_b.reshape(1, D))

    ap = _sc_aggr(npad, h, src, dst, w1.reshape(E), z_rows)
    out = _post(True, 2, ap, sp, h, nt2,
                lin1_0_w.T, lin1_0_b.reshape(1, D), lin1_1_w.T, lin1_1_b.reshape(1, D),
                ln1_g.reshape(1, D), ln1_b.reshape(1, D),
                fc_w.T, fc_b.reshape(1, D))
    return out
